# baseline (speedup 1.0000x reference)
import functools

import jax
import jax.numpy as jnp
from jax import lax
from jax.experimental import pallas as pl
from jax.experimental.pallas import tpu as pltpu
from jax.experimental.pallas import tpu_sc as plsc

N = 10000
E = 320000
G = 128
H = 64
DE = 16
DEPTH = 2
EPS = 1e-5

BN = 2000
BE = 2000
NB_N = N // BN
NB_E = E // BE

NC = 2
NS = 16
NW = NC * NS
EPT = E // NW
CH = 80
NCH = EPT // CH
SPT = 640
NSP = SPT * NS

_f32 = jnp.float32


def _iota_eq(idx, rows, axis):
    return (lax.broadcasted_iota(jnp.int32, rows, axis) == idx).astype(_f32)


def _sc_take1d(table, idx3, dtype):
    mesh = plsc.VectorSubcoreMesh(core_axis_name="c", subcore_axis_name="s")

    @functools.partial(
        pl.kernel,
        out_type=jax.ShapeDtypeStruct((E,), dtype),
        mesh=mesh,
        compiler_params=pltpu.CompilerParams(use_tc_tiling_on_sc=False),
        scratch_types=[
            pltpu.VMEM((NCH, CH), jnp.int32),
            pltpu.VMEM((CH,), dtype),
            pltpu.VMEM((CH,), dtype),
            pltpu.SemaphoreType.DMA,
            pltpu.SemaphoreType.DMA,
            pltpu.SemaphoreType.DMA,
            pltpu.SemaphoreType.DMA,
        ],
    )
    def k(batch_hbm, row3_hbm, out_hbm, ridx2, bufa, bufb, sga, sgb, swa, swb):
        wid = lax.axis_index("s") * NC + lax.axis_index("c")
        base = wid * EPT
        pltpu.sync_copy(row3_hbm.at[wid], ridx2)

        def issue(c, buf, sem):
            pltpu.async_copy(batch_hbm.at[ridx2.at[c]], buf, sem)

        def wait_g(buf, sem):
            pltpu.make_async_copy(batch_hbm.at[ridx2.at[0]], buf, sem).wait()

        def wout(c, buf, sem):
            pltpu.async_copy(buf, out_hbm.at[pl.ds(base + c * CH, CH)], sem)

        def wait_w(buf, sem):
            pltpu.make_async_copy(buf, out_hbm.at[pl.ds(0, CH)], sem).wait()

        issue(0, bufa, sga)
        issue(1, bufb, sgb)

        def body(kk, _):
            c0 = 2 * kk
            wait_g(bufa, sga)
            wout(c0 - 2, bufa, swa)
            wait_g(bufb, sgb)
            wout(c0 - 1, bufb, swb)
            wait_w(bufa, swa)
            issue(c0, bufa, sga)
            wait_w(bufb, swb)
            issue(c0 + 1, bufb, sgb)
            return 0

        lax.fori_loop(1, (NCH - 1) // 2, body, 0)
        wait_g(bufa, sga)
        wout(NCH - 3, bufa, swa)
        wait_g(bufb, sgb)
        wout(NCH - 2, bufb, swb)
        wait_w(bufa, swa)
        issue(NCH - 1, bufa, sga)
        wait_g(bufa, sga)
        wout(NCH - 1, bufa, swa)
        wait_w(bufa, swa)
        wait_w(bufb, swb)

    return k(table, idx3)


def _sc_gather(row3, col3, t, xb):
    mesh = plsc.VectorSubcoreMesh(core_axis_name="c", subcore_axis_name="s")

    @functools.partial(
        pl.kernel,
        out_type=[jax.ShapeDtypeStruct((E, H), _f32),
                  jax.ShapeDtypeStruct((E, H), _f32)],
        mesh=mesh,
        compiler_params=pltpu.CompilerParams(use_tc_tiling_on_sc=False),
        scratch_types=[
            pltpu.VMEM((NCH, CH), jnp.int32),
            pltpu.VMEM((NCH, CH), jnp.int32),
            pltpu.VMEM((CH, H), _f32),
            pltpu.VMEM((CH, H), _f32),
            pltpu.VMEM((CH, H), _f32),
            pltpu.VMEM((CH, H), _f32),
            pltpu.SemaphoreType.DMA,
            pltpu.SemaphoreType.DMA,
            pltpu.SemaphoreType.DMA,
            pltpu.SemaphoreType.DMA,
        ],
    )
    def k(row3_hbm, col3_hbm, t_hbm, xb_hbm, ot_hbm, oxb_hbm,
          ridx2, cidx2, ta, tb, xa, xb_, sga, sgb, swa, swb):
        wid = lax.axis_index("s") * NC + lax.axis_index("c")
        base = wid * EPT
        pltpu.sync_copy(row3_hbm.at[wid], ridx2)
        pltpu.sync_copy(col3_hbm.at[wid], cidx2)

        def issue(c, bt, bx, sem):
            pltpu.async_copy(t_hbm.at[ridx2.at[c]], bt, sem)
            pltpu.async_copy(xb_hbm.at[cidx2.at[c]], bx, sem)

        def wait_g(bt, bx, sem):
            pltpu.make_async_copy(t_hbm.at[ridx2.at[0]], bt, sem).wait()
            pltpu.make_async_copy(xb_hbm.at[cidx2.at[0]], bx, sem).wait()

        def wout(c, bt, bx, sem):
            off = base + c * CH
            pltpu.async_copy(bt, ot_hbm.at[pl.ds(off, CH)], sem)
            pltpu.async_copy(bx, oxb_hbm.at[pl.ds(off, CH)], sem)

        def wait_w(bt, bx, sem):
            pltpu.make_async_copy(bt, ot_hbm.at[pl.ds(0, CH)], sem).wait()
            pltpu.make_async_copy(bx, oxb_hbm.at[pl.ds(0, CH)], sem).wait()

        issue(0, ta, xa, sga)
        issue(1, tb, xb_, sgb)

        def body(kk, _):
            c0 = 2 * kk
            wait_g(ta, xa, sga)
            wout(c0 - 2, ta, xa, swa)
            wait_g(tb, xb_, sgb)
            wout(c0 - 1, tb, xb_, swb)
            wait_w(ta, xa, swa)
            issue(c0, ta, xa, sga)
            wait_w(tb, xb_, swb)
            issue(c0 + 1, tb, xb_, sgb)
            return 0

        lax.fori_loop(1, (NCH - 1) // 2, body, 0)
        wait_g(ta, xa, sga)
        wout(NCH - 3, ta, xa, swa)
        wait_g(tb, xb_, sgb)
        wout(NCH - 2, tb, xb_, swb)
        wait_w(ta, xa, swa)
        issue(NCH - 1, ta, xa, sga)
        wait_g(ta, xa, sga)
        wout(NCH - 1, ta, xa, swa)
        wait_w(ta, xa, swa)
        wait_w(tb, xb_, swb)

    return k(row3, col3, t, xb)


def _sc_scatter(row3, col3, xnorm, enew, zrows):
    mesh = plsc.VectorSubcoreMesh(core_axis_name="c", subcore_axis_name="s")

    @functools.partial(
        pl.kernel,
        out_type=[jax.ShapeDtypeStruct((NC, NSP, H), _f32),
                  jax.ShapeDtypeStruct((NC, NSP, H), _f32)],
        mesh=mesh,
        compiler_params=pltpu.CompilerParams(use_tc_tiling_on_sc=False),
        scratch_types=[
            pltpu.VMEM((NCH, CH), jnp.int32),
            pltpu.VMEM((NCH, CH), jnp.int32),
            pltpu.VMEM((CH, H), _f32),
            pltpu.VMEM((CH, H), _f32),
            pltpu.VMEM((CH, H), _f32),
            pltpu.VMEM((CH, H), _f32),
            pltpu.VMEM_SHARED((NSP, H), _f32),
            pltpu.VMEM_SHARED((NSP, H), _f32),
            pltpu.SemaphoreType.DMA,
            pltpu.SemaphoreType.DMA,
            pltpu.SemaphoreType.DMA,
            pltpu.SemaphoreType.DMA,
        ],
    )
    def k(row3_hbm, col3_hbm, xn_hbm, en_hbm, z_hbm, oa_hbm, ob_hbm,
          ridx2, cidx2, aa, ab, ba, bb, spa, spb, sga, sgb, ssa, ssb):
        cid = lax.axis_index("c")
        sid = lax.axis_index("s")
        wid = sid * NC + cid
        base = wid * EPT
        pltpu.sync_copy(z_hbm, spa.at[pl.ds(sid * SPT, SPT)])
        pltpu.sync_copy(z_hbm, spb.at[pl.ds(sid * SPT, SPT)])
        pltpu.sync_copy(row3_hbm.at[wid], ridx2)
        pltpu.sync_copy(col3_hbm.at[wid], cidx2)
        plsc.subcore_barrier()

        def issue(c, bxn, ben, sem):
            pltpu.async_copy(xn_hbm.at[ridx2.at[c]], bxn, sem)
            pltpu.async_copy(en_hbm.at[pl.ds(base + c * CH, CH)], ben, sem)

        def wait_g(bxn, ben, sem):
            pltpu.make_async_copy(xn_hbm.at[ridx2.at[0]], bxn, sem).wait()
            pltpu.make_async_copy(en_hbm.at[pl.ds(0, CH)], ben, sem).wait()

        def scat(c, bxn, ben, sem):
            pltpu.async_copy(bxn, spa.at[cidx2.at[c]], sem, add=True)
            pltpu.async_copy(ben, spb.at[cidx2.at[c]], sem, add=True)

        def wait_s(bxn, ben, sem):
            pltpu.make_async_copy(bxn, spa.at[cidx2.at[0]], sem).wait()
            pltpu.make_async_copy(ben, spb.at[cidx2.at[0]], sem).wait()

        issue(0, aa, ba, sga)
        issue(1, ab, bb, sgb)

        def body(kk, _):
            c0 = 2 * kk
            wait_g(aa, ba, sga)
            scat(c0 - 2, aa, ba, ssa)
            wait_g(ab, bb, sgb)
            scat(c0 - 1, ab, bb, ssb)
            wait_s(aa, ba, ssa)
            issue(c0, aa, ba, sga)
            wait_s(ab, bb, ssb)
            issue(c0 + 1, ab, bb, sgb)
            return 0

        lax.fori_loop(1, (NCH - 1) // 2, body, 0)
        wait_g(aa, ba, sga)
        scat(NCH - 3, aa, ba, ssa)
        wait_g(ab, bb, sgb)
        scat(NCH - 2, ab, bb, ssb)
        wait_s(aa, ba, ssa)
        issue(NCH - 1, aa, ba, sga)
        wait_g(aa, ba, sga)
        scat(NCH - 1, aa, ba, ssa)
        wait_s(aa, ba, ssa)
        wait_s(ab, bb, ssb)
        plsc.subcore_barrier()
        pltpu.sync_copy(spa.at[pl.ds(sid * SPT, SPT)],
                        oa_hbm.at[cid, pl.ds(sid * SPT, SPT)])
        pltpu.sync_copy(spb.at[pl.ds(sid * SPT, SPT)],
                        ob_hbm.at[cid, pl.ds(sid * SPT, SPT)])

    return k(row3, col3, xnorm, enew, zrows)


def _dot(a, b):
    return jnp.dot(a, b, preferred_element_type=_f32)


def _dot_hi(a, b):
    return jnp.dot(a, b, preferred_element_type=_f32,
                   precision=lax.Precision.HIGHEST)


def _gsum(oht, v):
    hi = v.astype(jnp.bfloat16).astype(_f32)
    r = v - hi
    mid = r.astype(jnp.bfloat16).astype(_f32)
    return _dot(oht, hi) + (_dot(oht, mid) + _dot(oht, r - mid))


def _enc_x(x, batch_r, W1, b1, W2, b2):
    def body(x_ref, bidx_ref, w1, bb1, w2, bb2, o_x1, o_s, o_q, o_c):
        i = pl.program_id(0)
        xx = x_ref[...]
        h = jnp.maximum(_dot(xx, w1[...]) + bb1[...], 0.0)
        x1 = _dot(h, w2[...]) + bb2[...]
        o_x1[...] = x1
        bidx = bidx_ref[0, 0, :]
        oht = _iota_eq(bidx[None, :], (G, BN), 0)

        @pl.when(i == 0)
        def _():
            o_s[...] = jnp.zeros_like(o_s)
            o_q[...] = jnp.zeros_like(o_q)
            o_c[...] = jnp.zeros_like(o_c)

        o_s[...] += _gsum(oht, x1)
        o_q[...] += _gsum(oht, x1 * x1)
        o_c[...] += _dot(oht, jnp.ones((BN, H), _f32))

    return pl.pallas_call(
        body,
        grid=(NB_N,),
        in_specs=[
            pl.BlockSpec((BN, 128), lambda i: (i, 0)),
            pl.BlockSpec((1, 1, BN), lambda i: (i, 0, 0)),
            pl.BlockSpec((128, H), lambda i: (0, 0)),
            pl.BlockSpec((1, H), lambda i: (0, 0)),
            pl.BlockSpec((H, H), lambda i: (0, 0)),
            pl.BlockSpec((1, H), lambda i: (0, 0)),
        ],
        out_specs=[
            pl.BlockSpec((BN, H), lambda i: (i, 0)),
            pl.BlockSpec((G, H), lambda i: (0, 0)),
            pl.BlockSpec((G, H), lambda i: (0, 0)),
            pl.BlockSpec((G, H), lambda i: (0, 0)),
        ],
        out_shape=[
            jax.ShapeDtypeStruct((N, H), _f32),
            jax.ShapeDtypeStruct((G, H), _f32),
            jax.ShapeDtypeStruct((G, H), _f32),
            jax.ShapeDtypeStruct((G, H), _f32),
        ],
    )(x, batch_r, W1, b1, W2, b2)


def _enc_e(eattr, brow_r, W1, b1, W2, b2):
    def body(e_ref, bidx_ref, w1, bb1, w2, bb2, o_e0, o_sraw, o_s, o_q, o_c):
        i = pl.program_id(0)
        er = e_ref[...]
        h = jnp.maximum(_dot(er, w1[...]) + bb1[...], 0.0)
        e0 = _dot(h, w2[...]) + bb2[...]
        o_e0[...] = e0
        bidx = bidx_ref[0, 0, :]
        oht = _iota_eq(bidx[None, :], (G, BE), 0)

        @pl.when(i == 0)
        def _():
            o_sraw[...] = jnp.zeros_like(o_sraw)
            o_s[...] = jnp.zeros_like(o_s)
            o_q[...] = jnp.zeros_like(o_q)
            o_c[...] = jnp.zeros_like(o_c)

        o_sraw[...] += _gsum(oht, er)
        o_s[...] += _gsum(oht, e0)
        o_q[...] += _gsum(oht, e0 * e0)
        o_c[...] += _dot(oht, jnp.ones((BE, H), _f32))

    return pl.pallas_call(
        body,
        grid=(NB_E,),
        in_specs=[
            pl.BlockSpec((BE, DE), lambda i: (i, 0)),
            pl.BlockSpec((1, 1, BE), lambda i: (i, 0, 0)),
            pl.BlockSpec((DE, H), lambda i: (0, 0)),
            pl.BlockSpec((1, H), lambda i: (0, 0)),
            pl.BlockSpec((H, H), lambda i: (0, 0)),
            pl.BlockSpec((1, H), lambda i: (0, 0)),
        ],
        out_specs=[
            pl.BlockSpec((BE, H), lambda i: (i, 0)),
            pl.BlockSpec((G, DE), lambda i: (0, 0)),
            pl.BlockSpec((G, H), lambda i: (0, 0)),
            pl.BlockSpec((G, H), lambda i: (0, 0)),
            pl.BlockSpec((G, H), lambda i: (0, 0)),
        ],
        out_shape=[
            jax.ShapeDtypeStruct((E, H), _f32),
            jax.ShapeDtypeStruct((G, DE), _f32),
            jax.ShapeDtypeStruct((G, H), _f32),
            jax.ShapeDtypeStruct((G, H), _f32),
            jax.ShapeDtypeStruct((G, H), _f32),
        ],
    )(eattr, brow_r, W1, b1, W2, b2)


def _bn_fold_expr(gw, bw, S, Q, svec_col, total):
    m = _dot_hi(svec_col.T, S) / total
    exx = _dot_hi((svec_col * svec_col).T, Q) / total
    v = exx - m * m
    scale = gw * lax.rsqrt(v + EPS)
    shift = bw - m * scale
    return scale, shift


def _glob0(S_raw, cnte, cntn, S_x, Q_x, S_e, Q_e,
           gx, bx, ge, be_, gu, bu, W1, b1, W2, b2):
    def body(sraw, ce_, cn_, sx, qx, se, qe, gxr, bxr, ger, ber, gur, bur,
             w1, bb1, w2, bb2, o_u, o_scal, o_svec):
        dege = ce_[...][:, 0:1]
        degn = cn_[...][:, 0:1]
        cee = jnp.maximum(dege, 1.0)
        cnn = jnp.maximum(degn, 1.0)
        u0 = sraw[...] / cee
        h = jnp.maximum(_dot(u0, w1[...]) + bb1[...], 0.0)
        u1 = _dot(h, w2[...]) + bb2[...]
        m = jnp.mean(u1, axis=0, keepdims=True)
        v = jnp.mean(u1 * u1, axis=0, keepdims=True) - m * m
        o_u[...] = (u1 - m) * lax.rsqrt(v + EPS) * gur[...] + bur[...]
        s_n = jnp.where(degn > 0, lax.rsqrt(jnp.maximum(degn, 1e-30)), 0.0)
        s_e = jnp.where(dege > 0, lax.rsqrt(jnp.maximum(dege, 1e-30)), 0.0)
        sc_x, sh_x = _bn_fold_expr(gxr[...], bxr[...], sx[...], qx[...], s_n, float(N))
        sc_e, sh_e = _bn_fold_expr(ger[...], ber[...], se[...], qe[...], s_e, float(E))
        o_scal[...] = jnp.concatenate(
            [sc_x, sh_x, sc_e, sh_e, jnp.zeros((4, H), _f32)], axis=0)
        o_svec[...] = jnp.concatenate(
            [s_n.reshape(1, G), s_e.reshape(1, G),
             (1.0 / cee).reshape(1, G), (1.0 / cnn).reshape(1, G),
             jnp.zeros((4, G), _f32)], axis=0)

    return pl.pallas_call(
        body,
        out_shape=[
            jax.ShapeDtypeStruct((G, H), _f32),
            jax.ShapeDtypeStruct((8, H), _f32),
            jax.ShapeDtypeStruct((8, G), _f32),
        ],
    )(S_raw, cnte, cntn, S_x, Q_x, S_e, Q_e,
      gx, bx, ge, be_, gu, bu, W1, b1, W2, b2)


def _tables(xraw, batch_r, scal, svec, u, W1a, W1b, W1d, b1):
    def body(x_ref, bidx_ref, scal_ref, svec_ref, u_ref, wa, wb, wd, bb1,
             o_xn, o_t, o_xb):
        bidx = bidx_ref[0, 0, :]
        oh = _iota_eq(bidx[:, None], (BN, G), 1)
        s_row = jnp.sum(oh * svec_ref[0:1, :], axis=1, keepdims=True)
        xn = x_ref[...] * s_row * scal_ref[0:1, :] + scal_ref[1:2, :]
        o_xn[...] = xn
        ug = _dot(u_ref[...], wd[...])
        o_t[...] = _dot(xn, wa[...]) + _dot_hi(oh, ug) + bb1[...]
        o_xb[...] = _dot(xn, wb[...])

    return pl.pallas_call(
        body,
        grid=(NB_N,),
        in_specs=[
            pl.BlockSpec((BN, H), lambda i: (i, 0)),
            pl.BlockSpec((1, 1, BN), lambda i: (i, 0, 0)),
            pl.BlockSpec((8, H), lambda i: (0, 0)),
            pl.BlockSpec((8, G), lambda i: (0, 0)),
            pl.BlockSpec((G, H), lambda i: (0, 0)),
            pl.BlockSpec((H, H), lambda i: (0, 0)),
            pl.BlockSpec((H, H), lambda i: (0, 0)),
            pl.BlockSpec((H, H), lambda i: (0, 0)),
            pl.BlockSpec((1, H), lambda i: (0, 0)),
        ],
        out_specs=[
            pl.BlockSpec((BN, H), lambda i: (i, 0)),
            pl.BlockSpec((BN, H), lambda i: (i, 0)),
            pl.BlockSpec((BN, H), lambda i: (i, 0)),
        ],
        out_shape=[
            jax.ShapeDtypeStruct((N, H), _f32),
            jax.ShapeDtypeStruct((N, H), _f32),
            jax.ShapeDtypeStruct((N, H), _f32),
        ],
    )(xraw, batch_r, scal, svec, u, W1a, W1b, W1d, b1)


def _edge(t_row, xb_col, eprev, brow_r, sedge_r, scal, W1c, W2, b2):
    def body(t_ref, xb_ref, ep_ref, bidx_ref, se_ref, scal_ref, w1c, w2, bb2,
             o_en, o_s, o_q):
        i = pl.program_id(0)
        bidx = bidx_ref[0, 0, :]
        s_edge = se_ref[0, 0, :].reshape(BE, 1)
        sc_e = scal_ref[2:3, :]
        sh_e = scal_ref[3:4, :]
        e_bn = ep_ref[...] * s_edge * sc_e + sh_e
        ec = _dot(e_bn, w1c[...])
        pre = t_ref[...] + xb_ref[...] + ec
        h = jnp.maximum(pre, 0.0)
        en = _dot(h, w2[...]) + bb2[...]
        o_en[...] = en
        oht = _iota_eq(bidx[None, :], (G, BE), 0)

        @pl.when(i == 0)
        def _():
            o_s[...] = jnp.zeros_like(o_s)
            o_q[...] = jnp.zeros_like(o_q)

        o_s[...] += _gsum(oht, en)
        o_q[...] += _gsum(oht, en * en)

    return pl.pallas_call(
        body,
        grid=(NB_E,),
        in_specs=[
            pl.BlockSpec((BE, H), lambda i: (i, 0)),
            pl.BlockSpec((BE, H), lambda i: (i, 0)),
            pl.BlockSpec((BE, H), lambda i: (i, 0)),
            pl.BlockSpec((1, 1, BE), lambda i: (i, 0, 0)),
            pl.BlockSpec((1, 1, BE), lambda i: (i, 0, 0)),
            pl.BlockSpec((8, H), lambda i: (0, 0)),
            pl.BlockSpec((H, H), lambda i: (0, 0)),
            pl.BlockSpec((H, H), lambda i: (0, 0)),
            pl.BlockSpec((1, H), lambda i: (0, 0)),
        ],
        out_specs=[
            pl.BlockSpec((BE, H), lambda i: (i, 0)),
            pl.BlockSpec((G, H), lambda i: (0, 0)),
            pl.BlockSpec((G, H), lambda i: (0, 0)),
        ],
        out_shape=[
            jax.ShapeDtypeStruct((E, H), _f32),
            jax.ShapeDtypeStruct((G, H), _f32),
            jax.ShapeDtypeStruct((G, H), _f32),
        ],
    )(t_row, xb_col, eprev, brow_r, sedge_r, scal, W1c, W2, b2)


def _node(aggA, aggB, xnorm, batch_r, u, W11, b11, W12, b12, V1, bV1, V2, bV2):
    def body(aa_ref, ab_ref, x_ref, bidx_ref, u_ref, w11, bb11, w12, bb12,
             v1, bbv1, v2, bbv2, o_xn, o_s, o_q):
        i = pl.program_id(0)
        agga = aa_ref[0] + aa_ref[1]
        aggb = ab_ref[0] + ab_ref[1]
        w11v = w11[...]
        hn = jnp.maximum(_dot(agga, w11v[:H]) + _dot(aggb, w11v[H:]) + bb11[...], 0.0)
        hn = _dot(hn, w12[...]) + bb12[...]
        bidx = bidx_ref[0, 0, :]
        oh = _iota_eq(bidx[:, None], (BN, G), 1)
        ub = _dot_hi(oh, u_ref[...])
        v1v = v1[...]
        z = jnp.maximum(_dot(x_ref[...], v1v[:H]) + _dot(hn, v1v[H:2 * H])
                        + _dot(ub, v1v[2 * H:]) + bbv1[...], 0.0)
        xn = _dot(z, v2[...]) + bbv2[...]
        o_xn[...] = xn
        oht = _iota_eq(bidx[None, :], (G, BN), 0)

        @pl.when(i == 0)
        def _():
            o_s[...] = jnp.zeros_like(o_s)
            o_q[...] = jnp.zeros_like(o_q)

        o_s[...] += _gsum(oht, xn)
        o_q[...] += _gsum(oht, xn * xn)

    return pl.pallas_call(
        body,
        grid=(NB_N,),
        in_specs=[
            pl.BlockSpec((NC, BN, H), lambda i: (0, i, 0)),
            pl.BlockSpec((NC, BN, H), lambda i: (0, i, 0)),
            pl.BlockSpec((BN, H), lambda i: (i, 0)),
            pl.BlockSpec((1, 1, BN), lambda i: (i, 0, 0)),
            pl.BlockSpec((G, H), lambda i: (0, 0)),
            pl.BlockSpec((2 * H, H), lambda i: (0, 0)),
            pl.BlockSpec((1, H), lambda i: (0, 0)),
            pl.BlockSpec((H, H), lambda i: (0, 0)),
            pl.BlockSpec((1, H), lambda i: (0, 0)),
            pl.BlockSpec((3 * H, H), lambda i: (0, 0)),
            pl.BlockSpec((1, H), lambda i: (0, 0)),
            pl.BlockSpec((H, H), lambda i: (0, 0)),
            pl.BlockSpec((1, H), lambda i: (0, 0)),
        ],
        out_specs=[
            pl.BlockSpec((BN, H), lambda i: (i, 0)),
            pl.BlockSpec((G, H), lambda i: (0, 0)),
            pl.BlockSpec((G, H), lambda i: (0, 0)),
        ],
        out_shape=[
            jax.ShapeDtypeStruct((N, H), _f32),
            jax.ShapeDtypeStruct((G, H), _f32),
            jax.ShapeDtypeStruct((G, H), _f32),
        ],
    )(aggA, aggB, xnorm, batch_r, u, W11, b11, W12, b12, V1, bV1, V2, bV2)


def _glob_layer(u, S_e, Q_e, S_xn, Q_xn, svec,
                gx, bx, ge, be_, gu, bu, GW1, Gb1, GW2, Gb2):
    def body(u_ref, se, qe, sxn, qxn, svec_ref, gxr, bxr, ger, ber, gur, bur,
             gw1, gb1, gw2, gb2, o_u, o_scal):
        s_n = svec_ref[0:1, :].reshape(G, 1)
        s_e = svec_ref[1:2, :].reshape(G, 1)
        inv_ce = svec_ref[2:3, :].reshape(G, 1)
        inv_cn = svec_ref[3:4, :].reshape(G, 1)
        edge_info = se[...] * inv_ce
        node_info = sxn[...] * inv_cn
        gw1v = gw1[...]
        h = jnp.maximum(_dot(u_ref[...], gw1v[:H]) + _dot(node_info, gw1v[H:2 * H])
                        + _dot(edge_info, gw1v[2 * H:]) + gb1[...], 0.0)
        un = _dot(h, gw2[...]) + gb2[...]
        m = jnp.mean(un, axis=0, keepdims=True)
        v = jnp.mean(un * un, axis=0, keepdims=True) - m * m
        o_u[...] = (un - m) * lax.rsqrt(v + EPS) * gur[...] + bur[...]
        sc_x, sh_x = _bn_fold_expr(gxr[...], bxr[...], sxn[...], qxn[...], s_n, float(N))
        sc_e, sh_e = _bn_fold_expr(ger[...], ber[...], se[...], qe[...], s_e, float(E))
        o_scal[...] = jnp.concatenate(
            [sc_x, sh_x, sc_e, sh_e, jnp.zeros((4, H), _f32)], axis=0)

    return pl.pallas_call(
        body,
        out_shape=[
            jax.ShapeDtypeStruct((G, H), _f32),
            jax.ShapeDtypeStruct((8, H), _f32),
        ],
    )(u, S_e, Q_e, S_xn, Q_xn, svec, gx, bx, ge, be_, gu, bu, GW1, Gb1, GW2, Gb2)


def _final(u, W1, b1, W2, b2):
    def body(u_ref, w1, bb1, w2, bb2, o):
        h = jnp.maximum(_dot(u_ref[...], w1[...]) + bb1[...], 0.0)
        o[...] = _dot(h, w2[...]) + bb2[...]

    return pl.pallas_call(
        body,
        out_shape=jax.ShapeDtypeStruct((G, 1), _f32),
    )(u, W1, b1, W2, b2)


def kernel(x, edge_attr, params, edge_index, batch):
    row3 = edge_index[0].reshape(NW, NCH, CH)
    col3 = edge_index[1].reshape(NW, NCH, CH)
    brow = _sc_take1d(batch, row3, jnp.int32)
    batch_r = batch.reshape(NB_N, 1, BN)
    brow_r = brow.reshape(NB_E, 1, BE)
    zrows = jnp.zeros((SPT, H), _f32)

    def r1(v):
        return v.reshape(1, -1)

    pn = params["mlp_node"]
    pe = params["mlp_edge"]
    pg = params["mlp_global"]
    x1, S_x, Q_x, cntn = _enc_x(x, batch_r, pn["W1"], r1(pn["b1"]),
                                pn["W2"], r1(pn["b2"]))
    e0, S_raw, S_e, Q_e, cnte = _enc_e(edge_attr, brow_r, pe["W1"], r1(pe["b1"]),
                                       pe["W2"], r1(pe["b2"]))
    bnx = params["bn_node"][DEPTH]
    bne = params["bn_edge"][DEPTH]
    bnu = params["bn_global"][DEPTH]
    u, scal, svec = _glob0(S_raw, cnte, cntn, S_x, Q_x, S_e, Q_e,
                           r1(bnx["g"]), r1(bnx["b"]), r1(bne["g"]), r1(bne["b"]),
                           r1(bnu["g"]), r1(bnu["b"]),
                           pg["W1"], r1(pg["b1"]), pg["W2"], r1(pg["b2"]))

    brow3 = brow.reshape(NW, NCH, CH)
    s_edge = _sc_take1d(svec[1], brow3, _f32)
    sedge_r = s_edge.reshape(NB_E, 1, BE)

    eprev = e0
    xraw = x1
    for i in range(DEPTH):
        lp = params["layers"][i]
        em = lp["edge_mlp"]
        W1 = em["W1"]
        xnorm, t, xb = _tables(xraw, batch_r, scal, svec, u,
                               W1[:H], W1[H:2 * H], W1[3 * H:], r1(em["b1"]))
        t_row, xb_col = _sc_gather(row3, col3, t, xb)
        enew, S_e, Q_e = _edge(t_row, xb_col, eprev, brow_r, sedge_r, scal,
                               W1[2 * H:3 * H], em["W2"], r1(em["b2"]))
        aggA, aggB = _sc_scatter(row3, col3, xnorm, enew, zrows)
        nm1 = lp["node_mlp_1"]
        nm2 = lp["node_mlp_2"]
        xnew, S_xn, Q_xn = _node(aggA, aggB, xnorm, batch_r, u,
                                 nm1["W1"], r1(nm1["b1"]), nm1["W2"], r1(nm1["b2"]),
                                 nm2["W1"], r1(nm2["b1"]), nm2["W2"], r1(nm2["b2"]))
        bnx = params["bn_node"][i]
        bne = params["bn_edge"][i]
        bnu = params["bn_global"][i]
        gm = lp["global_mlp"]
        u, scal = _glob_layer(u, S_e, Q_e, S_xn, Q_xn, svec,
                              r1(bnx["g"]), r1(bnx["b"]), r1(bne["g"]), r1(bne["b"]),
                              r1(bnu["g"]), r1(bnu["b"]),
                              gm["W1"], r1(gm["b1"]), gm["W2"], r1(gm["b2"]))
        eprev = enew
        xraw = xnew

    m1 = params["mlp1"]
    return _final(u, m1["W1"], r1(m1["b1"]), m1["W2"], r1(m1["b2"]))


# speedup vs baseline: 1.1678x; 1.1678x over previous
import functools

import jax
import jax.numpy as jnp
from jax import lax
from jax.experimental import pallas as pl
from jax.experimental.pallas import tpu as pltpu
from jax.experimental.pallas import tpu_sc as plsc

N = 10000
E = 320000
G = 128
H = 64
DE = 16
DEPTH = 2
EPS = 1e-5

BN = 2000
BE = 2000
NB_N = N // BN
NB_E = E // BE

NC = 2
NS = 16
NW = NC * NS
EPT = E // NW
CH = 80
NCH = EPT // CH
SPT = 640
NSP = SPT * NS

_f32 = jnp.float32


def _iota_eq(idx, rows, axis):
    return (lax.broadcasted_iota(jnp.int32, rows, axis) == idx).astype(_f32)


def _sc_take1d(table, idx3, dtype):
    mesh = plsc.VectorSubcoreMesh(core_axis_name="c", subcore_axis_name="s")

    @functools.partial(
        pl.kernel,
        out_type=jax.ShapeDtypeStruct((E,), dtype),
        mesh=mesh,
        compiler_params=pltpu.CompilerParams(use_tc_tiling_on_sc=False),
        scratch_types=[
            pltpu.VMEM((NCH, CH), jnp.int32),
            pltpu.VMEM((CH,), dtype),
            pltpu.VMEM((CH,), dtype),
            pltpu.SemaphoreType.DMA,
            pltpu.SemaphoreType.DMA,
            pltpu.SemaphoreType.DMA,
            pltpu.SemaphoreType.DMA,
        ],
    )
    def k(batch_hbm, row3_hbm, out_hbm, ridx2, bufa, bufb, sga, sgb, swa, swb):
        wid = lax.axis_index("s") * NC + lax.axis_index("c")
        base = wid * EPT
        pltpu.sync_copy(row3_hbm.at[wid], ridx2)

        def issue(c, buf, sem):
            pltpu.async_copy(batch_hbm.at[ridx2.at[c]], buf, sem)

        def wait_g(buf, sem):
            pltpu.make_async_copy(batch_hbm.at[ridx2.at[0]], buf, sem).wait()

        def wout(c, buf, sem):
            pltpu.async_copy(buf, out_hbm.at[pl.ds(base + c * CH, CH)], sem)

        def wait_w(buf, sem):
            pltpu.make_async_copy(buf, out_hbm.at[pl.ds(0, CH)], sem).wait()

        issue(0, bufa, sga)
        issue(1, bufb, sgb)

        def body(kk, _):
            c0 = 2 * kk
            wait_g(bufa, sga)
            wout(c0 - 2, bufa, swa)
            wait_g(bufb, sgb)
            wout(c0 - 1, bufb, swb)
            wait_w(bufa, swa)
            issue(c0, bufa, sga)
            wait_w(bufb, swb)
            issue(c0 + 1, bufb, sgb)
            return 0

        lax.fori_loop(1, (NCH - 1) // 2, body, 0)
        wait_g(bufa, sga)
        wout(NCH - 3, bufa, swa)
        wait_g(bufb, sgb)
        wout(NCH - 2, bufb, swb)
        wait_w(bufa, swa)
        issue(NCH - 1, bufa, sga)
        wait_g(bufa, sga)
        wout(NCH - 1, bufa, swa)
        wait_w(bufa, swa)
        wait_w(bufb, swb)

    return k(table, idx3)


def _sc_gather(row3, col3, t, xb):
    mesh = plsc.VectorSubcoreMesh(core_axis_name="c", subcore_axis_name="s")

    @functools.partial(
        pl.kernel,
        out_type=[jax.ShapeDtypeStruct((E, H), _f32),
                  jax.ShapeDtypeStruct((E, H), _f32)],
        mesh=mesh,
        compiler_params=pltpu.CompilerParams(use_tc_tiling_on_sc=False),
        scratch_types=[
            pltpu.VMEM((NCH, CH), jnp.int32),
            pltpu.VMEM((NCH, CH), jnp.int32),
            pltpu.VMEM((CH, H), _f32),
            pltpu.VMEM((CH, H), _f32),
            pltpu.VMEM((CH, H), _f32),
            pltpu.VMEM((CH, H), _f32),
            pltpu.SemaphoreType.DMA,
            pltpu.SemaphoreType.DMA,
            pltpu.SemaphoreType.DMA,
            pltpu.SemaphoreType.DMA,
        ],
    )
    def k(row3_hbm, col3_hbm, t_hbm, xb_hbm, ot_hbm, oxb_hbm,
          ridx2, cidx2, ta, tb, xa, xb_, sga, sgb, swa, swb):
        wid = lax.axis_index("s") * NC + lax.axis_index("c")
        base = wid * EPT
        pltpu.sync_copy(row3_hbm.at[wid], ridx2)
        pltpu.sync_copy(col3_hbm.at[wid], cidx2)

        def issue(c, bt, bx, sem):
            pltpu.async_copy(t_hbm.at[ridx2.at[c]], bt, sem)
            pltpu.async_copy(xb_hbm.at[cidx2.at[c]], bx, sem)

        def wait_g(bt, bx, sem):
            pltpu.make_async_copy(t_hbm.at[ridx2.at[0]], bt, sem).wait()
            pltpu.make_async_copy(xb_hbm.at[cidx2.at[0]], bx, sem).wait()

        def wout(c, bt, bx, sem):
            off = base + c * CH
            pltpu.async_copy(bt, ot_hbm.at[pl.ds(off, CH)], sem)
            pltpu.async_copy(bx, oxb_hbm.at[pl.ds(off, CH)], sem)

        def wait_w(bt, bx, sem):
            pltpu.make_async_copy(bt, ot_hbm.at[pl.ds(0, CH)], sem).wait()
            pltpu.make_async_copy(bx, oxb_hbm.at[pl.ds(0, CH)], sem).wait()

        issue(0, ta, xa, sga)
        issue(1, tb, xb_, sgb)

        def body(kk, _):
            c0 = 2 * kk
            wait_g(ta, xa, sga)
            wout(c0 - 2, ta, xa, swa)
            wait_g(tb, xb_, sgb)
            wout(c0 - 1, tb, xb_, swb)
            wait_w(ta, xa, swa)
            issue(c0, ta, xa, sga)
            wait_w(tb, xb_, swb)
            issue(c0 + 1, tb, xb_, sgb)
            return 0

        lax.fori_loop(1, (NCH - 1) // 2, body, 0)
        wait_g(ta, xa, sga)
        wout(NCH - 3, ta, xa, swa)
        wait_g(tb, xb_, sgb)
        wout(NCH - 2, tb, xb_, swb)
        wait_w(ta, xa, swa)
        issue(NCH - 1, ta, xa, sga)
        wait_g(ta, xa, sga)
        wout(NCH - 1, ta, xa, swa)
        wait_w(ta, xa, swa)
        wait_w(tb, xb_, swb)

    return k(row3, col3, t, xb)


def _sc_scatter(row3, col3, xnorm, enew, zrows):
    mesh = plsc.VectorSubcoreMesh(core_axis_name="c", subcore_axis_name="s")

    @functools.partial(
        pl.kernel,
        out_type=[jax.ShapeDtypeStruct((NC, NSP, H), _f32),
                  jax.ShapeDtypeStruct((NC, NSP, H), _f32)],
        mesh=mesh,
        compiler_params=pltpu.CompilerParams(use_tc_tiling_on_sc=False),
        scratch_types=[
            pltpu.VMEM((NCH, CH), jnp.int32),
            pltpu.VMEM((NCH, CH), jnp.int32),
            pltpu.VMEM((CH, H), _f32),
            pltpu.VMEM((CH, H), _f32),
            pltpu.VMEM((CH, H), _f32),
            pltpu.VMEM((CH, H), _f32),
            pltpu.VMEM_SHARED((NSP, H), _f32),
            pltpu.VMEM_SHARED((NSP, H), _f32),
            pltpu.SemaphoreType.DMA,
            pltpu.SemaphoreType.DMA,
            pltpu.SemaphoreType.DMA,
            pltpu.SemaphoreType.DMA,
        ],
    )
    def k(row3_hbm, col3_hbm, xn_hbm, en_hbm, z_hbm, oa_hbm, ob_hbm,
          ridx2, cidx2, aa, ab, ba, bb, spa, spb, sga, sgb, ssa, ssb):
        cid = lax.axis_index("c")
        sid = lax.axis_index("s")
        wid = sid * NC + cid
        base = wid * EPT
        pltpu.sync_copy(z_hbm, spa.at[pl.ds(sid * SPT, SPT)])
        pltpu.sync_copy(z_hbm, spb.at[pl.ds(sid * SPT, SPT)])
        pltpu.sync_copy(row3_hbm.at[wid], ridx2)
        pltpu.sync_copy(col3_hbm.at[wid], cidx2)
        plsc.subcore_barrier()

        def issue(c, bxn, ben, sem):
            pltpu.async_copy(xn_hbm.at[ridx2.at[c]], bxn, sem)
            pltpu.async_copy(en_hbm.at[pl.ds(base + c * CH, CH)], ben, sem)

        def wait_g(bxn, ben, sem):
            pltpu.make_async_copy(xn_hbm.at[ridx2.at[0]], bxn, sem).wait()
            pltpu.make_async_copy(en_hbm.at[pl.ds(0, CH)], ben, sem).wait()

        def scat(c, bxn, ben, sem):
            pltpu.async_copy(bxn, spa.at[cidx2.at[c]], sem, add=True)
            pltpu.async_copy(ben, spb.at[cidx2.at[c]], sem, add=True)

        def wait_s(bxn, ben, sem):
            pltpu.make_async_copy(bxn, spa.at[cidx2.at[0]], sem).wait()
            pltpu.make_async_copy(ben, spb.at[cidx2.at[0]], sem).wait()

        issue(0, aa, ba, sga)
        issue(1, ab, bb, sgb)

        def body(kk, _):
            c0 = 2 * kk
            wait_g(aa, ba, sga)
            scat(c0 - 2, aa, ba, ssa)
            wait_g(ab, bb, sgb)
            scat(c0 - 1, ab, bb, ssb)
            wait_s(aa, ba, ssa)
            issue(c0, aa, ba, sga)
            wait_s(ab, bb, ssb)
            issue(c0 + 1, ab, bb, sgb)
            return 0

        lax.fori_loop(1, (NCH - 1) // 2, body, 0)
        wait_g(aa, ba, sga)
        scat(NCH - 3, aa, ba, ssa)
        wait_g(ab, bb, sgb)
        scat(NCH - 2, ab, bb, ssb)
        wait_s(aa, ba, ssa)
        issue(NCH - 1, aa, ba, sga)
        wait_g(aa, ba, sga)
        scat(NCH - 1, aa, ba, ssa)
        wait_s(aa, ba, ssa)
        wait_s(ab, bb, ssb)
        plsc.subcore_barrier()
        pltpu.sync_copy(spa.at[pl.ds(sid * SPT, SPT)],
                        oa_hbm.at[cid, pl.ds(sid * SPT, SPT)])
        pltpu.sync_copy(spb.at[pl.ds(sid * SPT, SPT)],
                        ob_hbm.at[cid, pl.ds(sid * SPT, SPT)])

    return k(row3, col3, xnorm, enew, zrows)


def _dot(a, b):
    return jnp.dot(a, b, preferred_element_type=_f32)


def _dot_hi(a, b):
    return jnp.dot(a, b, preferred_element_type=_f32,
                   precision=lax.Precision.HIGHEST)


def _gsum(oht, v):
    hi = v.astype(jnp.bfloat16).astype(_f32)
    r = v - hi
    mid = r.astype(jnp.bfloat16).astype(_f32)
    return _dot(oht, hi) + (_dot(oht, mid) + _dot(oht, r - mid))


def _enc_x(x, batch_r, W1, b1, W2, b2):
    def body(x_ref, bidx_ref, w1, bb1, w2, bb2, o_x1, o_s, o_q, o_c):
        i = pl.program_id(0)
        xx = x_ref[...]
        h = jnp.maximum(_dot(xx, w1[...]) + bb1[...], 0.0)
        x1 = _dot(h, w2[...]) + bb2[...]
        o_x1[...] = x1
        bidx = bidx_ref[0, 0, :]
        oht = _iota_eq(bidx[None, :], (G, BN), 0)

        @pl.when(i == 0)
        def _():
            o_s[...] = jnp.zeros_like(o_s)
            o_q[...] = jnp.zeros_like(o_q)
            o_c[...] = jnp.zeros_like(o_c)

        o_s[...] += _gsum(oht, x1)
        o_q[...] += _gsum(oht, x1 * x1)
        o_c[...] += _dot(oht, jnp.ones((BN, H), _f32))

    return pl.pallas_call(
        body,
        grid=(NB_N,),
        in_specs=[
            pl.BlockSpec((BN, 128), lambda i: (i, 0)),
            pl.BlockSpec((1, 1, BN), lambda i: (i, 0, 0)),
            pl.BlockSpec((128, H), lambda i: (0, 0)),
            pl.BlockSpec((1, H), lambda i: (0, 0)),
            pl.BlockSpec((H, H), lambda i: (0, 0)),
            pl.BlockSpec((1, H), lambda i: (0, 0)),
        ],
        out_specs=[
            pl.BlockSpec((BN, H), lambda i: (i, 0)),
            pl.BlockSpec((G, H), lambda i: (0, 0)),
            pl.BlockSpec((G, H), lambda i: (0, 0)),
            pl.BlockSpec((G, H), lambda i: (0, 0)),
        ],
        out_shape=[
            jax.ShapeDtypeStruct((N, H), _f32),
            jax.ShapeDtypeStruct((G, H), _f32),
            jax.ShapeDtypeStruct((G, H), _f32),
            jax.ShapeDtypeStruct((G, H), _f32),
        ],
    )(x, batch_r, W1, b1, W2, b2)


def _enc_e(eattr, brow_r, W1, b1, W2, b2):
    def body(e_ref, bidx_ref, w1, bb1, w2, bb2, o_e0, o_sraw, o_s, o_q, o_c):
        i = pl.program_id(0)
        er = e_ref[...]
        h = jnp.maximum(_dot(er, w1[...]) + bb1[...], 0.0)
        e0 = _dot(h, w2[...]) + bb2[...]
        o_e0[...] = e0
        bidx = bidx_ref[0, 0, :]
        oht = _iota_eq(bidx[None, :], (G, BE), 0)

        @pl.when(i == 0)
        def _():
            o_sraw[...] = jnp.zeros_like(o_sraw)
            o_s[...] = jnp.zeros_like(o_s)
            o_q[...] = jnp.zeros_like(o_q)
            o_c[...] = jnp.zeros_like(o_c)

        o_sraw[...] += _gsum(oht, er)
        o_s[...] += _gsum(oht, e0)
        o_q[...] += _gsum(oht, e0 * e0)
        o_c[...] += _dot(oht, jnp.ones((BE, H), _f32))

    return pl.pallas_call(
        body,
        grid=(NB_E,),
        in_specs=[
            pl.BlockSpec((BE, DE), lambda i: (i, 0)),
            pl.BlockSpec((1, 1, BE), lambda i: (i, 0, 0)),
            pl.BlockSpec((DE, H), lambda i: (0, 0)),
            pl.BlockSpec((1, H), lambda i: (0, 0)),
            pl.BlockSpec((H, H), lambda i: (0, 0)),
            pl.BlockSpec((1, H), lambda i: (0, 0)),
        ],
        out_specs=[
            pl.BlockSpec((BE, H), lambda i: (i, 0)),
            pl.BlockSpec((G, DE), lambda i: (0, 0)),
            pl.BlockSpec((G, H), lambda i: (0, 0)),
            pl.BlockSpec((G, H), lambda i: (0, 0)),
            pl.BlockSpec((G, H), lambda i: (0, 0)),
        ],
        out_shape=[
            jax.ShapeDtypeStruct((E, H), _f32),
            jax.ShapeDtypeStruct((G, DE), _f32),
            jax.ShapeDtypeStruct((G, H), _f32),
            jax.ShapeDtypeStruct((G, H), _f32),
            jax.ShapeDtypeStruct((G, H), _f32),
        ],
    )(eattr, brow_r, W1, b1, W2, b2)


def _bn_fold_expr(gw, bw, S, Q, svec_col, total):
    m = _dot_hi(svec_col.T, S) / total
    exx = _dot_hi((svec_col * svec_col).T, Q) / total
    v = exx - m * m
    scale = gw * lax.rsqrt(v + EPS)
    shift = bw - m * scale
    return scale, shift


def _glob0(S_raw, cnte, cntn, S_x, Q_x, S_e, Q_e,
           gx, bx, ge, be_, gu, bu, W1, b1, W2, b2):
    def body(sraw, ce_, cn_, sx, qx, se, qe, gxr, bxr, ger, ber, gur, bur,
             w1, bb1, w2, bb2, o_u, o_scal, o_svec):
        dege = ce_[...][:, 0:1]
        degn = cn_[...][:, 0:1]
        cee = jnp.maximum(dege, 1.0)
        cnn = jnp.maximum(degn, 1.0)
        u0 = sraw[...] / cee
        h = jnp.maximum(_dot(u0, w1[...]) + bb1[...], 0.0)
        u1 = _dot(h, w2[...]) + bb2[...]
        m = jnp.mean(u1, axis=0, keepdims=True)
        v = jnp.mean(u1 * u1, axis=0, keepdims=True) - m * m
        o_u[...] = (u1 - m) * lax.rsqrt(v + EPS) * gur[...] + bur[...]
        s_n = jnp.where(degn > 0, lax.rsqrt(jnp.maximum(degn, 1e-30)), 0.0)
        s_e = jnp.where(dege > 0, lax.rsqrt(jnp.maximum(dege, 1e-30)), 0.0)
        sc_x, sh_x = _bn_fold_expr(gxr[...], bxr[...], sx[...], qx[...], s_n, float(N))
        sc_e, sh_e = _bn_fold_expr(ger[...], ber[...], se[...], qe[...], s_e, float(E))
        o_scal[...] = jnp.concatenate(
            [sc_x, sh_x, sc_e, sh_e, jnp.zeros((4, H), _f32)], axis=0)
        o_svec[...] = jnp.concatenate(
            [s_n.reshape(1, G), s_e.reshape(1, G),
             (1.0 / cee).reshape(1, G), (1.0 / cnn).reshape(1, G),
             jnp.zeros((4, G), _f32)], axis=0)

    return pl.pallas_call(
        body,
        out_shape=[
            jax.ShapeDtypeStruct((G, H), _f32),
            jax.ShapeDtypeStruct((8, H), _f32),
            jax.ShapeDtypeStruct((8, G), _f32),
        ],
    )(S_raw, cnte, cntn, S_x, Q_x, S_e, Q_e,
      gx, bx, ge, be_, gu, bu, W1, b1, W2, b2)


def _tables(xraw, batch_r, scal, svec, u, W1a, W1b, W1d, b1):
    def body(x_ref, bidx_ref, scal_ref, svec_ref, u_ref, wa, wb, wd, bb1,
             o_xn, o_t, o_xb):
        bidx = bidx_ref[0, 0, :]
        oh = _iota_eq(bidx[:, None], (BN, G), 1)
        s_row = jnp.sum(oh * svec_ref[0:1, :], axis=1, keepdims=True)
        xn = x_ref[...] * s_row * scal_ref[0:1, :] + scal_ref[1:2, :]
        o_xn[...] = xn
        ug = _dot(u_ref[...], wd[...])
        o_t[...] = _dot(xn, wa[...]) + _dot_hi(oh, ug) + bb1[...]
        o_xb[...] = _dot(xn, wb[...])

    return pl.pallas_call(
        body,
        grid=(NB_N,),
        in_specs=[
            pl.BlockSpec((BN, H), lambda i: (i, 0)),
            pl.BlockSpec((1, 1, BN), lambda i: (i, 0, 0)),
            pl.BlockSpec((8, H), lambda i: (0, 0)),
            pl.BlockSpec((8, G), lambda i: (0, 0)),
            pl.BlockSpec((G, H), lambda i: (0, 0)),
            pl.BlockSpec((H, H), lambda i: (0, 0)),
            pl.BlockSpec((H, H), lambda i: (0, 0)),
            pl.BlockSpec((H, H), lambda i: (0, 0)),
            pl.BlockSpec((1, H), lambda i: (0, 0)),
        ],
        out_specs=[
            pl.BlockSpec((BN, H), lambda i: (i, 0)),
            pl.BlockSpec((BN, H), lambda i: (i, 0)),
            pl.BlockSpec((BN, H), lambda i: (i, 0)),
        ],
        out_shape=[
            jax.ShapeDtypeStruct((N, H), _f32),
            jax.ShapeDtypeStruct((N, H), _f32),
            jax.ShapeDtypeStruct((N, H), _f32),
        ],
    )(xraw, batch_r, scal, svec, u, W1a, W1b, W1d, b1)


def _edge(t_row, xb_col, eprev, brow_r, svec, scal, W1c, W2, b2):
    def body(t_ref, xb_ref, ep_ref, bidx_ref, svec_ref, scal_ref, w1c, w2, bb2,
             o_en, o_s, o_q):
        i = pl.program_id(0)
        bidx = bidx_ref[0, 0, :]
        oh = _iota_eq(bidx[:, None], (BE, G), 1)
        s_edge = jnp.sum(oh * svec_ref[1:2, :], axis=1, keepdims=True)
        sc_e = scal_ref[2:3, :]
        sh_e = scal_ref[3:4, :]
        e_bn = ep_ref[...] * s_edge * sc_e + sh_e
        ec = _dot(e_bn, w1c[...])
        pre = t_ref[...] + xb_ref[...] + ec
        h = jnp.maximum(pre, 0.0)
        en = _dot(h, w2[...]) + bb2[...]
        o_en[...] = en
        oht = _iota_eq(bidx[None, :], (G, BE), 0)

        @pl.when(i == 0)
        def _():
            o_s[...] = jnp.zeros_like(o_s)
            o_q[...] = jnp.zeros_like(o_q)

        o_s[...] += _gsum(oht, en)
        o_q[...] += _gsum(oht, en * en)

    return pl.pallas_call(
        body,
        grid=(NB_E,),
        in_specs=[
            pl.BlockSpec((BE, H), lambda i: (i, 0)),
            pl.BlockSpec((BE, H), lambda i: (i, 0)),
            pl.BlockSpec((BE, H), lambda i: (i, 0)),
            pl.BlockSpec((1, 1, BE), lambda i: (i, 0, 0)),
            pl.BlockSpec((8, G), lambda i: (0, 0)),
            pl.BlockSpec((8, H), lambda i: (0, 0)),
            pl.BlockSpec((H, H), lambda i: (0, 0)),
            pl.BlockSpec((H, H), lambda i: (0, 0)),
            pl.BlockSpec((1, H), lambda i: (0, 0)),
        ],
        out_specs=[
            pl.BlockSpec((BE, H), lambda i: (i, 0)),
            pl.BlockSpec((G, H), lambda i: (0, 0)),
            pl.BlockSpec((G, H), lambda i: (0, 0)),
        ],
        out_shape=[
            jax.ShapeDtypeStruct((E, H), _f32),
            jax.ShapeDtypeStruct((G, H), _f32),
            jax.ShapeDtypeStruct((G, H), _f32),
        ],
    )(t_row, xb_col, eprev, brow_r, svec, scal, W1c, W2, b2)


def _node(aggA, aggB, xnorm, batch_r, u, W11, b11, W12, b12, V1, bV1, V2, bV2):
    def body(aa_ref, ab_ref, x_ref, bidx_ref, u_ref, w11, bb11, w12, bb12,
             v1, bbv1, v2, bbv2, o_xn, o_s, o_q):
        i = pl.program_id(0)
        agga = aa_ref[0] + aa_ref[1]
        aggb = ab_ref[0] + ab_ref[1]
        w11v = w11[...]
        hn = jnp.maximum(_dot(agga, w11v[:H]) + _dot(aggb, w11v[H:]) + bb11[...], 0.0)
        hn = _dot(hn, w12[...]) + bb12[...]
        bidx = bidx_ref[0, 0, :]
        oh = _iota_eq(bidx[:, None], (BN, G), 1)
        ub = _dot_hi(oh, u_ref[...])
        v1v = v1[...]
        z = jnp.maximum(_dot(x_ref[...], v1v[:H]) + _dot(hn, v1v[H:2 * H])
                        + _dot(ub, v1v[2 * H:]) + bbv1[...], 0.0)
        xn = _dot(z, v2[...]) + bbv2[...]
        o_xn[...] = xn
        oht = _iota_eq(bidx[None, :], (G, BN), 0)

        @pl.when(i == 0)
        def _():
            o_s[...] = jnp.zeros_like(o_s)
            o_q[...] = jnp.zeros_like(o_q)

        o_s[...] += _gsum(oht, xn)
        o_q[...] += _gsum(oht, xn * xn)

    return pl.pallas_call(
        body,
        grid=(NB_N,),
        in_specs=[
            pl.BlockSpec((NC, BN, H), lambda i: (0, i, 0)),
            pl.BlockSpec((NC, BN, H), lambda i: (0, i, 0)),
            pl.BlockSpec((BN, H), lambda i: (i, 0)),
            pl.BlockSpec((1, 1, BN), lambda i: (i, 0, 0)),
            pl.BlockSpec((G, H), lambda i: (0, 0)),
            pl.BlockSpec((2 * H, H), lambda i: (0, 0)),
            pl.BlockSpec((1, H), lambda i: (0, 0)),
            pl.BlockSpec((H, H), lambda i: (0, 0)),
            pl.BlockSpec((1, H), lambda i: (0, 0)),
            pl.BlockSpec((3 * H, H), lambda i: (0, 0)),
            pl.BlockSpec((1, H), lambda i: (0, 0)),
            pl.BlockSpec((H, H), lambda i: (0, 0)),
            pl.BlockSpec((1, H), lambda i: (0, 0)),
        ],
        out_specs=[
            pl.BlockSpec((BN, H), lambda i: (i, 0)),
            pl.BlockSpec((G, H), lambda i: (0, 0)),
            pl.BlockSpec((G, H), lambda i: (0, 0)),
        ],
        out_shape=[
            jax.ShapeDtypeStruct((N, H), _f32),
            jax.ShapeDtypeStruct((G, H), _f32),
            jax.ShapeDtypeStruct((G, H), _f32),
        ],
    )(aggA, aggB, xnorm, batch_r, u, W11, b11, W12, b12, V1, bV1, V2, bV2)


def _glob_layer(u, S_e, Q_e, S_xn, Q_xn, svec,
                gx, bx, ge, be_, gu, bu, GW1, Gb1, GW2, Gb2):
    def body(u_ref, se, qe, sxn, qxn, svec_ref, gxr, bxr, ger, ber, gur, bur,
             gw1, gb1, gw2, gb2, o_u, o_scal):
        s_n = svec_ref[0:1, :].reshape(G, 1)
        s_e = svec_ref[1:2, :].reshape(G, 1)
        inv_ce = svec_ref[2:3, :].reshape(G, 1)
        inv_cn = svec_ref[3:4, :].reshape(G, 1)
        edge_info = se[...] * inv_ce
        node_info = sxn[...] * inv_cn
        gw1v = gw1[...]
        h = jnp.maximum(_dot(u_ref[...], gw1v[:H]) + _dot(node_info, gw1v[H:2 * H])
                        + _dot(edge_info, gw1v[2 * H:]) + gb1[...], 0.0)
        un = _dot(h, gw2[...]) + gb2[...]
        m = jnp.mean(un, axis=0, keepdims=True)
        v = jnp.mean(un * un, axis=0, keepdims=True) - m * m
        o_u[...] = (un - m) * lax.rsqrt(v + EPS) * gur[...] + bur[...]
        sc_x, sh_x = _bn_fold_expr(gxr[...], bxr[...], sxn[...], qxn[...], s_n, float(N))
        sc_e, sh_e = _bn_fold_expr(ger[...], ber[...], se[...], qe[...], s_e, float(E))
        o_scal[...] = jnp.concatenate(
            [sc_x, sh_x, sc_e, sh_e, jnp.zeros((4, H), _f32)], axis=0)

    return pl.pallas_call(
        body,
        out_shape=[
            jax.ShapeDtypeStruct((G, H), _f32),
            jax.ShapeDtypeStruct((8, H), _f32),
        ],
    )(u, S_e, Q_e, S_xn, Q_xn, svec, gx, bx, ge, be_, gu, bu, GW1, Gb1, GW2, Gb2)


def _final(u, W1, b1, W2, b2):
    def body(u_ref, w1, bb1, w2, bb2, o):
        h = jnp.maximum(_dot(u_ref[...], w1[...]) + bb1[...], 0.0)
        o[...] = _dot(h, w2[...]) + bb2[...]

    return pl.pallas_call(
        body,
        out_shape=jax.ShapeDtypeStruct((G, 1), _f32),
    )(u, W1, b1, W2, b2)


def kernel(x, edge_attr, params, edge_index, batch):
    row3 = edge_index[0].reshape(NW, NCH, CH)
    col3 = edge_index[1].reshape(NW, NCH, CH)
    brow = _sc_take1d(batch, row3, jnp.int32)
    batch_r = batch.reshape(NB_N, 1, BN)
    brow_r = brow.reshape(NB_E, 1, BE)
    zrows = jnp.zeros((SPT, H), _f32)

    def r1(v):
        return v.reshape(1, -1)

    pn = params["mlp_node"]
    pe = params["mlp_edge"]
    pg = params["mlp_global"]
    x1, S_x, Q_x, cntn = _enc_x(x, batch_r, pn["W1"], r1(pn["b1"]),
                                pn["W2"], r1(pn["b2"]))
    e0, S_raw, S_e, Q_e, cnte = _enc_e(edge_attr, brow_r, pe["W1"], r1(pe["b1"]),
                                       pe["W2"], r1(pe["b2"]))
    bnx = params["bn_node"][DEPTH]
    bne = params["bn_edge"][DEPTH]
    bnu = params["bn_global"][DEPTH]
    u, scal, svec = _glob0(S_raw, cnte, cntn, S_x, Q_x, S_e, Q_e,
                           r1(bnx["g"]), r1(bnx["b"]), r1(bne["g"]), r1(bne["b"]),
                           r1(bnu["g"]), r1(bnu["b"]),
                           pg["W1"], r1(pg["b1"]), pg["W2"], r1(pg["b2"]))

    eprev = e0
    xraw = x1
    for i in range(DEPTH):
        lp = params["layers"][i]
        em = lp["edge_mlp"]
        W1 = em["W1"]
        xnorm, t, xb = _tables(xraw, batch_r, scal, svec, u,
                               W1[:H], W1[H:2 * H], W1[3 * H:], r1(em["b1"]))
        t_row, xb_col = _sc_gather(row3, col3, t, xb)
        enew, S_e, Q_e = _edge(t_row, xb_col, eprev, brow_r, svec, scal,
                               W1[2 * H:3 * H], em["W2"], r1(em["b2"]))
        aggA, aggB = _sc_scatter(row3, col3, xnorm, enew, zrows)
        nm1 = lp["node_mlp_1"]
        nm2 = lp["node_mlp_2"]
        xnew, S_xn, Q_xn = _node(aggA, aggB, xnorm, batch_r, u,
                                 nm1["W1"], r1(nm1["b1"]), nm1["W2"], r1(nm1["b2"]),
                                 nm2["W1"], r1(nm2["b1"]), nm2["W2"], r1(nm2["b2"]))
        bnx = params["bn_node"][i]
        bne = params["bn_edge"][i]
        bnu = params["bn_global"][i]
        gm = lp["global_mlp"]
        u, scal = _glob_layer(u, S_e, Q_e, S_xn, Q_xn, svec,
                              r1(bnx["g"]), r1(bnx["b"]), r1(bne["g"]), r1(bne["b"]),
                              r1(bnu["g"]), r1(bnu["b"]),
                              gm["W1"], r1(gm["b1"]), gm["W2"], r1(gm["b2"]))
        eprev = enew
        xraw = xnew

    m1 = params["mlp1"]
    return _final(u, m1["W1"], r1(m1["b1"]), m1["W2"], r1(m1["b2"]))


# speedup vs baseline: 1.3324x; 1.1409x over previous
import functools

import jax
import jax.numpy as jnp
from jax import lax
from jax.experimental import pallas as pl
from jax.experimental.pallas import tpu as pltpu
from jax.experimental.pallas import tpu_sc as plsc

N = 10000
E = 320000
G = 128
H = 64
DE = 16
DEPTH = 2
EPS = 1e-5

BN = 2000
BE = 4000
NB_N = N // BN
NB_E = E // BE

NC = 2
NS = 16
NW = NC * NS
EPT = E // NW
CH = 80
NCH = EPT // CH
SPT = 640
NSP = SPT * NS

_f32 = jnp.float32


def _iota_eq(idx, rows, axis):
    return (lax.broadcasted_iota(jnp.int32, rows, axis) == idx).astype(_f32)


def _sc_take1d(table, idx3, dtype):
    mesh = plsc.VectorSubcoreMesh(core_axis_name="c", subcore_axis_name="s")

    @functools.partial(
        pl.kernel,
        out_type=jax.ShapeDtypeStruct((E,), dtype),
        mesh=mesh,
        compiler_params=pltpu.CompilerParams(use_tc_tiling_on_sc=False),
        scratch_types=[
            pltpu.VMEM((NCH, CH), jnp.int32),
            pltpu.VMEM((CH,), dtype),
            pltpu.VMEM((CH,), dtype),
            pltpu.SemaphoreType.DMA,
            pltpu.SemaphoreType.DMA,
            pltpu.SemaphoreType.DMA,
            pltpu.SemaphoreType.DMA,
        ],
    )
    def k(batch_hbm, row3_hbm, out_hbm, ridx2, bufa, bufb, sga, sgb, swa, swb):
        wid = lax.axis_index("s") * NC + lax.axis_index("c")
        base = wid * EPT
        pltpu.sync_copy(row3_hbm.at[wid], ridx2)

        def issue(c, buf, sem):
            pltpu.async_copy(batch_hbm.at[ridx2.at[c]], buf, sem)

        def wait_g(buf, sem):
            pltpu.make_async_copy(batch_hbm.at[ridx2.at[0]], buf, sem).wait()

        def wout(c, buf, sem):
            pltpu.async_copy(buf, out_hbm.at[pl.ds(base + c * CH, CH)], sem)

        def wait_w(buf, sem):
            pltpu.make_async_copy(buf, out_hbm.at[pl.ds(0, CH)], sem).wait()

        issue(0, bufa, sga)
        issue(1, bufb, sgb)

        def body(kk, _):
            c0 = 2 * kk
            wait_g(bufa, sga)
            wout(c0 - 2, bufa, swa)
            wait_g(bufb, sgb)
            wout(c0 - 1, bufb, swb)
            wait_w(bufa, swa)
            issue(c0, bufa, sga)
            wait_w(bufb, swb)
            issue(c0 + 1, bufb, sgb)
            return 0

        lax.fori_loop(1, (NCH - 1) // 2, body, 0)
        wait_g(bufa, sga)
        wout(NCH - 3, bufa, swa)
        wait_g(bufb, sgb)
        wout(NCH - 2, bufb, swb)
        wait_w(bufa, swa)
        issue(NCH - 1, bufa, sga)
        wait_g(bufa, sga)
        wout(NCH - 1, bufa, swa)
        wait_w(bufa, swa)
        wait_w(bufb, swb)

    return k(table, idx3)


def _sc_gather(row3, col3, t, xb):
    mesh = plsc.VectorSubcoreMesh(core_axis_name="c", subcore_axis_name="s")

    @functools.partial(
        pl.kernel,
        out_type=[jax.ShapeDtypeStruct((E, H), _f32),
                  jax.ShapeDtypeStruct((E, H), _f32)],
        mesh=mesh,
        compiler_params=pltpu.CompilerParams(use_tc_tiling_on_sc=False),
        scratch_types=[
            pltpu.VMEM((NCH, CH), jnp.int32),
            pltpu.VMEM((NCH, CH), jnp.int32),
            pltpu.VMEM((CH, H), _f32),
            pltpu.VMEM((CH, H), _f32),
            pltpu.VMEM((CH, H), _f32),
            pltpu.VMEM((CH, H), _f32),
            pltpu.SemaphoreType.DMA,
            pltpu.SemaphoreType.DMA,
            pltpu.SemaphoreType.DMA,
            pltpu.SemaphoreType.DMA,
        ],
    )
    def k(row3_hbm, col3_hbm, t_hbm, xb_hbm, ot_hbm, oxb_hbm,
          ridx2, cidx2, ta, tb, xa, xb_, sga, sgb, swa, swb):
        wid = lax.axis_index("s") * NC + lax.axis_index("c")
        base = wid * EPT
        pltpu.sync_copy(row3_hbm.at[wid], ridx2)
        pltpu.sync_copy(col3_hbm.at[wid], cidx2)

        def issue(c, bt, bx, sem):
            pltpu.async_copy(t_hbm.at[ridx2.at[c]], bt, sem)
            pltpu.async_copy(xb_hbm.at[cidx2.at[c]], bx, sem)

        def wait_g(bt, bx, sem):
            pltpu.make_async_copy(t_hbm.at[ridx2.at[0]], bt, sem).wait()
            pltpu.make_async_copy(xb_hbm.at[cidx2.at[0]], bx, sem).wait()

        def wout(c, bt, bx, sem):
            off = base + c * CH
            pltpu.async_copy(bt, ot_hbm.at[pl.ds(off, CH)], sem)
            pltpu.async_copy(bx, oxb_hbm.at[pl.ds(off, CH)], sem)

        def wait_w(bt, bx, sem):
            pltpu.make_async_copy(bt, ot_hbm.at[pl.ds(0, CH)], sem).wait()
            pltpu.make_async_copy(bx, oxb_hbm.at[pl.ds(0, CH)], sem).wait()

        issue(0, ta, xa, sga)
        issue(1, tb, xb_, sgb)

        def body(kk, _):
            c0 = 2 * kk
            wait_g(ta, xa, sga)
            wout(c0 - 2, ta, xa, swa)
            wait_g(tb, xb_, sgb)
            wout(c0 - 1, tb, xb_, swb)
            wait_w(ta, xa, swa)
            issue(c0, ta, xa, sga)
            wait_w(tb, xb_, swb)
            issue(c0 + 1, tb, xb_, sgb)
            return 0

        lax.fori_loop(1, (NCH - 1) // 2, body, 0)
        wait_g(ta, xa, sga)
        wout(NCH - 3, ta, xa, swa)
        wait_g(tb, xb_, sgb)
        wout(NCH - 2, tb, xb_, swb)
        wait_w(ta, xa, swa)
        issue(NCH - 1, ta, xa, sga)
        wait_g(ta, xa, sga)
        wout(NCH - 1, ta, xa, swa)
        wait_w(ta, xa, swa)
        wait_w(tb, xb_, swb)

    return k(row3, col3, t, xb)


def _sc_scatter(row3, col3, xnorm, enew, zrows):
    mesh = plsc.VectorSubcoreMesh(core_axis_name="c", subcore_axis_name="s")

    @functools.partial(
        pl.kernel,
        out_type=[jax.ShapeDtypeStruct((NC, NSP, H), _f32),
                  jax.ShapeDtypeStruct((NC, NSP, H), _f32)],
        mesh=mesh,
        compiler_params=pltpu.CompilerParams(use_tc_tiling_on_sc=False),
        scratch_types=[
            pltpu.VMEM((NCH, CH), jnp.int32),
            pltpu.VMEM((NCH, CH), jnp.int32),
            pltpu.VMEM((CH, H), _f32),
            pltpu.VMEM((CH, H), _f32),
            pltpu.VMEM((CH, H), _f32),
            pltpu.VMEM((CH, H), _f32),
            pltpu.VMEM_SHARED((NSP, H), _f32),
            pltpu.VMEM_SHARED((NSP, H), _f32),
            pltpu.SemaphoreType.DMA,
            pltpu.SemaphoreType.DMA,
            pltpu.SemaphoreType.DMA,
            pltpu.SemaphoreType.DMA,
        ],
    )
    def k(row3_hbm, col3_hbm, xn_hbm, en_hbm, z_hbm, oa_hbm, ob_hbm,
          ridx2, cidx2, aa, ab, ba, bb, spa, spb, sga, sgb, ssa, ssb):
        cid = lax.axis_index("c")
        sid = lax.axis_index("s")
        wid = sid * NC + cid
        base = wid * EPT
        pltpu.sync_copy(z_hbm, spa.at[pl.ds(sid * SPT, SPT)])
        pltpu.sync_copy(z_hbm, spb.at[pl.ds(sid * SPT, SPT)])
        pltpu.sync_copy(row3_hbm.at[wid], ridx2)
        pltpu.sync_copy(col3_hbm.at[wid], cidx2)
        plsc.subcore_barrier()

        def issue(c, bxn, ben, sem):
            pltpu.async_copy(xn_hbm.at[ridx2.at[c]], bxn, sem)
            pltpu.async_copy(en_hbm.at[pl.ds(base + c * CH, CH)], ben, sem)

        def wait_g(bxn, ben, sem):
            pltpu.make_async_copy(xn_hbm.at[ridx2.at[0]], bxn, sem).wait()
            pltpu.make_async_copy(en_hbm.at[pl.ds(0, CH)], ben, sem).wait()

        def scat(c, bxn, ben, sem):
            pltpu.async_copy(bxn, spa.at[cidx2.at[c]], sem, add=True)
            pltpu.async_copy(ben, spb.at[cidx2.at[c]], sem, add=True)

        def wait_s(bxn, ben, sem):
            pltpu.make_async_copy(bxn, spa.at[cidx2.at[0]], sem).wait()
            pltpu.make_async_copy(ben, spb.at[cidx2.at[0]], sem).wait()

        issue(0, aa, ba, sga)
        issue(1, ab, bb, sgb)

        def body(kk, _):
            c0 = 2 * kk
            wait_g(aa, ba, sga)
            scat(c0 - 2, aa, ba, ssa)
            wait_g(ab, bb, sgb)
            scat(c0 - 1, ab, bb, ssb)
            wait_s(aa, ba, ssa)
            issue(c0, aa, ba, sga)
            wait_s(ab, bb, ssb)
            issue(c0 + 1, ab, bb, sgb)
            return 0

        lax.fori_loop(1, (NCH - 1) // 2, body, 0)
        wait_g(aa, ba, sga)
        scat(NCH - 3, aa, ba, ssa)
        wait_g(ab, bb, sgb)
        scat(NCH - 2, ab, bb, ssb)
        wait_s(aa, ba, ssa)
        issue(NCH - 1, aa, ba, sga)
        wait_g(aa, ba, sga)
        scat(NCH - 1, aa, ba, ssa)
        wait_s(aa, ba, ssa)
        wait_s(ab, bb, ssb)
        plsc.subcore_barrier()
        pltpu.sync_copy(spa.at[pl.ds(sid * SPT, SPT)],
                        oa_hbm.at[cid, pl.ds(sid * SPT, SPT)])
        pltpu.sync_copy(spb.at[pl.ds(sid * SPT, SPT)],
                        ob_hbm.at[cid, pl.ds(sid * SPT, SPT)])

    return k(row3, col3, xnorm, enew, zrows)


def _dot(a, b):
    return jnp.dot(a, b, preferred_element_type=_f32)


def _dot_hi(a, b):
    return jnp.dot(a, b, preferred_element_type=_f32,
                   precision=lax.Precision.HIGHEST)


def _gsum(oht, v):
    hi = v.astype(jnp.bfloat16).astype(_f32)
    r = v - hi
    mid = r.astype(jnp.bfloat16).astype(_f32)
    return _dot(oht, hi) + (_dot(oht, mid) + _dot(oht, r - mid))


def _enc_x(x, batch_r, W1, b1, W2, b2):
    def body(x_ref, bidx_ref, w1, bb1, w2, bb2, o_x1, o_s, o_q, o_c):
        i = pl.program_id(0)
        xx = x_ref[...]
        h = jnp.maximum(_dot(xx, w1[...]) + bb1[...], 0.0)
        x1 = _dot(h, w2[...]) + bb2[...]
        o_x1[...] = x1
        bidx = bidx_ref[0, 0, :]
        oht = _iota_eq(bidx[None, :], (G, BN), 0)

        @pl.when(i == 0)
        def _():
            o_s[...] = jnp.zeros_like(o_s)
            o_q[...] = jnp.zeros_like(o_q)
            o_c[...] = jnp.zeros_like(o_c)

        o_s[...] += _gsum(oht, x1)
        o_q[...] += _gsum(oht, x1 * x1)
        o_c[...] += _dot(oht, jnp.ones((BN, H), _f32))

    return pl.pallas_call(
        body,
        grid=(NB_N,),
        in_specs=[
            pl.BlockSpec((BN, 128), lambda i: (i, 0)),
            pl.BlockSpec((1, 1, BN), lambda i: (i, 0, 0)),
            pl.BlockSpec((128, H), lambda i: (0, 0)),
            pl.BlockSpec((1, H), lambda i: (0, 0)),
            pl.BlockSpec((H, H), lambda i: (0, 0)),
            pl.BlockSpec((1, H), lambda i: (0, 0)),
        ],
        out_specs=[
            pl.BlockSpec((BN, H), lambda i: (i, 0)),
            pl.BlockSpec((G, H), lambda i: (0, 0)),
            pl.BlockSpec((G, H), lambda i: (0, 0)),
            pl.BlockSpec((G, H), lambda i: (0, 0)),
        ],
        out_shape=[
            jax.ShapeDtypeStruct((N, H), _f32),
            jax.ShapeDtypeStruct((G, H), _f32),
            jax.ShapeDtypeStruct((G, H), _f32),
            jax.ShapeDtypeStruct((G, H), _f32),
        ],
    )(x, batch_r, W1, b1, W2, b2)


def _enc_e(eattr, brow_r, W1, b1, W2, b2):
    def body(e_ref, bidx_ref, w1, bb1, w2, bb2, o_e0, o_sraw, o_s, o_q, o_c):
        i = pl.program_id(0)
        er = e_ref[...]
        h = jnp.maximum(_dot(er, w1[...]) + bb1[...], 0.0)
        e0 = _dot(h, w2[...]) + bb2[...]
        o_e0[...] = e0
        bidx = bidx_ref[0, 0, :]
        oht = _iota_eq(bidx[None, :], (G, BE), 0)

        @pl.when(i == 0)
        def _():
            o_sraw[...] = jnp.zeros_like(o_sraw)
            o_s[...] = jnp.zeros_like(o_s)
            o_q[...] = jnp.zeros_like(o_q)
            o_c[...] = jnp.zeros_like(o_c)

        o_sraw[...] += _gsum(oht, er)
        o_s[...] += _gsum(oht, e0)
        o_q[...] += _gsum(oht, e0 * e0)
        o_c[...] += _dot(oht, jnp.ones((BE, H), _f32))

    return pl.pallas_call(
        body,
        grid=(NB_E,),
        in_specs=[
            pl.BlockSpec((BE, DE), lambda i: (i, 0)),
            pl.BlockSpec((1, 1, BE), lambda i: (i, 0, 0)),
            pl.BlockSpec((DE, H), lambda i: (0, 0)),
            pl.BlockSpec((1, H), lambda i: (0, 0)),
            pl.BlockSpec((H, H), lambda i: (0, 0)),
            pl.BlockSpec((1, H), lambda i: (0, 0)),
        ],
        out_specs=[
            pl.BlockSpec((BE, H), lambda i: (i, 0)),
            pl.BlockSpec((G, DE), lambda i: (0, 0)),
            pl.BlockSpec((G, H), lambda i: (0, 0)),
            pl.BlockSpec((G, H), lambda i: (0, 0)),
            pl.BlockSpec((G, H), lambda i: (0, 0)),
        ],
        out_shape=[
            jax.ShapeDtypeStruct((E, H), _f32),
            jax.ShapeDtypeStruct((G, DE), _f32),
            jax.ShapeDtypeStruct((G, H), _f32),
            jax.ShapeDtypeStruct((G, H), _f32),
            jax.ShapeDtypeStruct((G, H), _f32),
        ],
    )(eattr, brow_r, W1, b1, W2, b2)


def _bn_fold_expr(gw, bw, S, Q, svec_col, total):
    m = _dot_hi(svec_col.T, S) / total
    exx = _dot_hi((svec_col * svec_col).T, Q) / total
    v = exx - m * m
    scale = gw * lax.rsqrt(v + EPS)
    shift = bw - m * scale
    return scale, shift


def _glob0(S_raw, cnte, cntn, S_x, Q_x, S_e, Q_e,
           gx, bx, ge, be_, gu, bu, W1, b1, W2, b2):
    def body(sraw, ce_, cn_, sx, qx, se, qe, gxr, bxr, ger, ber, gur, bur,
             w1, bb1, w2, bb2, o_u, o_scal, o_svec):
        dege = ce_[...][:, 0:1]
        degn = cn_[...][:, 0:1]
        cee = jnp.maximum(dege, 1.0)
        cnn = jnp.maximum(degn, 1.0)
        u0 = sraw[...] / cee
        h = jnp.maximum(_dot(u0, w1[...]) + bb1[...], 0.0)
        u1 = _dot(h, w2[...]) + bb2[...]
        m = jnp.mean(u1, axis=0, keepdims=True)
        v = jnp.mean(u1 * u1, axis=0, keepdims=True) - m * m
        o_u[...] = (u1 - m) * lax.rsqrt(v + EPS) * gur[...] + bur[...]
        s_n = jnp.where(degn > 0, lax.rsqrt(jnp.maximum(degn, 1e-30)), 0.0)
        s_e = jnp.where(dege > 0, lax.rsqrt(jnp.maximum(dege, 1e-30)), 0.0)
        sc_x, sh_x = _bn_fold_expr(gxr[...], bxr[...], sx[...], qx[...], s_n, float(N))
        sc_e, sh_e = _bn_fold_expr(ger[...], ber[...], se[...], qe[...], s_e, float(E))
        o_scal[...] = jnp.concatenate(
            [sc_x, sh_x, sc_e, sh_e, jnp.zeros((4, H), _f32)], axis=0)
        o_svec[...] = jnp.concatenate(
            [s_n.reshape(1, G), s_e.reshape(1, G),
             (1.0 / cee).reshape(1, G), (1.0 / cnn).reshape(1, G),
             jnp.zeros((4, G), _f32)], axis=0)

    return pl.pallas_call(
        body,
        out_shape=[
            jax.ShapeDtypeStruct((G, H), _f32),
            jax.ShapeDtypeStruct((8, H), _f32),
            jax.ShapeDtypeStruct((8, G), _f32),
        ],
    )(S_raw, cnte, cntn, S_x, Q_x, S_e, Q_e,
      gx, bx, ge, be_, gu, bu, W1, b1, W2, b2)


def _tables(xraw, batch_r, scal, svec, u, W1a, W1b, W1d, b1):
    def body(x_ref, bidx_ref, scal_ref, svec_ref, u_ref, wa, wb, wd, bb1,
             o_xn, o_t, o_xb):
        bidx = bidx_ref[0, 0, :]
        oh = _iota_eq(bidx[:, None], (BN, G), 1)
        s_row = jnp.sum(oh * svec_ref[0:1, :], axis=1, keepdims=True)
        xn = x_ref[...] * s_row * scal_ref[0:1, :] + scal_ref[1:2, :]
        o_xn[...] = xn
        ug = _dot(u_ref[...], wd[...])
        o_t[...] = _dot(xn, wa[...]) + _dot_hi(oh, ug) + bb1[...]
        o_xb[...] = _dot(xn, wb[...])

    return pl.pallas_call(
        body,
        grid=(NB_N,),
        in_specs=[
            pl.BlockSpec((BN, H), lambda i: (i, 0)),
            pl.BlockSpec((1, 1, BN), lambda i: (i, 0, 0)),
            pl.BlockSpec((8, H), lambda i: (0, 0)),
            pl.BlockSpec((8, G), lambda i: (0, 0)),
            pl.BlockSpec((G, H), lambda i: (0, 0)),
            pl.BlockSpec((H, H), lambda i: (0, 0)),
            pl.BlockSpec((H, H), lambda i: (0, 0)),
            pl.BlockSpec((H, H), lambda i: (0, 0)),
            pl.BlockSpec((1, H), lambda i: (0, 0)),
        ],
        out_specs=[
            pl.BlockSpec((BN, H), lambda i: (i, 0)),
            pl.BlockSpec((BN, H), lambda i: (i, 0)),
            pl.BlockSpec((BN, H), lambda i: (i, 0)),
        ],
        out_shape=[
            jax.ShapeDtypeStruct((N, H), _f32),
            jax.ShapeDtypeStruct((N, H), _f32),
            jax.ShapeDtypeStruct((N, H), _f32),
        ],
    )(xraw, batch_r, scal, svec, u, W1a, W1b, W1d, b1)


def _edge(t_row, xb_col, eprev, brow_r, svec, scal, W1c, W2, b2):
    def body(t_ref, xb_ref, ep_ref, bidx_ref, svec_ref, scal_ref, w1c, w2, bb2,
             o_en, o_s, o_q):
        i = pl.program_id(0)
        bidx = bidx_ref[0, 0, :]
        oh = _iota_eq(bidx[:, None], (BE, G), 1)
        s_edge = jnp.sum(oh * svec_ref[1:2, :], axis=1, keepdims=True)
        sc_e = scal_ref[2:3, :]
        sh_e = scal_ref[3:4, :]
        e_bn = ep_ref[...] * s_edge * sc_e + sh_e
        ec = _dot(e_bn, w1c[...])
        pre = t_ref[...] + xb_ref[...] + ec
        h = jnp.maximum(pre, 0.0)
        en = _dot(h, w2[...]) + bb2[...]
        o_en[...] = en
        oht = _iota_eq(bidx[None, :], (G, BE), 0)

        @pl.when(i == 0)
        def _():
            o_s[...] = jnp.zeros_like(o_s)
            o_q[...] = jnp.zeros_like(o_q)

        o_s[...] += _gsum(oht, en)
        o_q[...] += _gsum(oht, en * en)

    return pl.pallas_call(
        body,
        grid=(NB_E,),
        in_specs=[
            pl.BlockSpec((BE, H), lambda i: (i, 0)),
            pl.BlockSpec((BE, H), lambda i: (i, 0)),
            pl.BlockSpec((BE, H), lambda i: (i, 0)),
            pl.BlockSpec((1, 1, BE), lambda i: (i, 0, 0)),
            pl.BlockSpec((8, G), lambda i: (0, 0)),
            pl.BlockSpec((8, H), lambda i: (0, 0)),
            pl.BlockSpec((H, H), lambda i: (0, 0)),
            pl.BlockSpec((H, H), lambda i: (0, 0)),
            pl.BlockSpec((1, H), lambda i: (0, 0)),
        ],
        out_specs=[
            pl.BlockSpec((BE, H), lambda i: (i, 0)),
            pl.BlockSpec((G, H), lambda i: (0, 0)),
            pl.BlockSpec((G, H), lambda i: (0, 0)),
        ],
        out_shape=[
            jax.ShapeDtypeStruct((E, H), _f32),
            jax.ShapeDtypeStruct((G, H), _f32),
            jax.ShapeDtypeStruct((G, H), _f32),
        ],
    )(t_row, xb_col, eprev, brow_r, svec, scal, W1c, W2, b2)


def _node(aggA, aggB, xnorm, batch_r, u, W11, b11, W12, b12, V1, bV1, V2, bV2):
    def body(aa_ref, ab_ref, x_ref, bidx_ref, u_ref, w11, bb11, w12, bb12,
             v1, bbv1, v2, bbv2, o_xn, o_s, o_q):
        i = pl.program_id(0)
        agga = aa_ref[0] + aa_ref[1]
        aggb = ab_ref[0] + ab_ref[1]
        w11v = w11[...]
        hn = jnp.maximum(_dot(agga, w11v[:H]) + _dot(aggb, w11v[H:]) + bb11[...], 0.0)
        hn = _dot(hn, w12[...]) + bb12[...]
        bidx = bidx_ref[0, 0, :]
        oh = _iota_eq(bidx[:, None], (BN, G), 1)
        ub = _dot_hi(oh, u_ref[...])
        v1v = v1[...]
        z = jnp.maximum(_dot(x_ref[...], v1v[:H]) + _dot(hn, v1v[H:2 * H])
                        + _dot(ub, v1v[2 * H:]) + bbv1[...], 0.0)
        xn = _dot(z, v2[...]) + bbv2[...]
        o_xn[...] = xn
        oht = _iota_eq(bidx[None, :], (G, BN), 0)

        @pl.when(i == 0)
        def _():
            o_s[...] = jnp.zeros_like(o_s)
            o_q[...] = jnp.zeros_like(o_q)

        o_s[...] += _gsum(oht, xn)
        o_q[...] += _gsum(oht, xn * xn)

    return pl.pallas_call(
        body,
        grid=(NB_N,),
        in_specs=[
            pl.BlockSpec((NC, BN, H), lambda i: (0, i, 0)),
            pl.BlockSpec((NC, BN, H), lambda i: (0, i, 0)),
            pl.BlockSpec((BN, H), lambda i: (i, 0)),
            pl.BlockSpec((1, 1, BN), lambda i: (i, 0, 0)),
            pl.BlockSpec((G, H), lambda i: (0, 0)),
            pl.BlockSpec((2 * H, H), lambda i: (0, 0)),
            pl.BlockSpec((1, H), lambda i: (0, 0)),
            pl.BlockSpec((H, H), lambda i: (0, 0)),
            pl.BlockSpec((1, H), lambda i: (0, 0)),
            pl.BlockSpec((3 * H, H), lambda i: (0, 0)),
            pl.BlockSpec((1, H), lambda i: (0, 0)),
            pl.BlockSpec((H, H), lambda i: (0, 0)),
            pl.BlockSpec((1, H), lambda i: (0, 0)),
        ],
        out_specs=[
            pl.BlockSpec((BN, H), lambda i: (i, 0)),
            pl.BlockSpec((G, H), lambda i: (0, 0)),
            pl.BlockSpec((G, H), lambda i: (0, 0)),
        ],
        out_shape=[
            jax.ShapeDtypeStruct((N, H), _f32),
            jax.ShapeDtypeStruct((G, H), _f32),
            jax.ShapeDtypeStruct((G, H), _f32),
        ],
    )(aggA, aggB, xnorm, batch_r, u, W11, b11, W12, b12, V1, bV1, V2, bV2)


def _glob_layer(u, S_e, Q_e, S_xn, Q_xn, svec,
                gx, bx, ge, be_, gu, bu, GW1, Gb1, GW2, Gb2):
    def body(u_ref, se, qe, sxn, qxn, svec_ref, gxr, bxr, ger, ber, gur, bur,
             gw1, gb1, gw2, gb2, o_u, o_scal):
        s_n = svec_ref[0:1, :].reshape(G, 1)
        s_e = svec_ref[1:2, :].reshape(G, 1)
        inv_ce = svec_ref[2:3, :].reshape(G, 1)
        inv_cn = svec_ref[3:4, :].reshape(G, 1)
        edge_info = se[...] * inv_ce
        node_info = sxn[...] * inv_cn
        gw1v = gw1[...]
        h = jnp.maximum(_dot(u_ref[...], gw1v[:H]) + _dot(node_info, gw1v[H:2 * H])
                        + _dot(edge_info, gw1v[2 * H:]) + gb1[...], 0.0)
        un = _dot(h, gw2[...]) + gb2[...]
        m = jnp.mean(un, axis=0, keepdims=True)
        v = jnp.mean(un * un, axis=0, keepdims=True) - m * m
        o_u[...] = (un - m) * lax.rsqrt(v + EPS) * gur[...] + bur[...]
        sc_x, sh_x = _bn_fold_expr(gxr[...], bxr[...], sxn[...], qxn[...], s_n, float(N))
        sc_e, sh_e = _bn_fold_expr(ger[...], ber[...], se[...], qe[...], s_e, float(E))
        o_scal[...] = jnp.concatenate(
            [sc_x, sh_x, sc_e, sh_e, jnp.zeros((4, H), _f32)], axis=0)

    return pl.pallas_call(
        body,
        out_shape=[
            jax.ShapeDtypeStruct((G, H), _f32),
            jax.ShapeDtypeStruct((8, H), _f32),
        ],
    )(u, S_e, Q_e, S_xn, Q_xn, svec, gx, bx, ge, be_, gu, bu, GW1, Gb1, GW2, Gb2)


def _final(u, W1, b1, W2, b2):
    def body(u_ref, w1, bb1, w2, bb2, o):
        h = jnp.maximum(_dot(u_ref[...], w1[...]) + bb1[...], 0.0)
        o[...] = _dot(h, w2[...]) + bb2[...]

    return pl.pallas_call(
        body,
        out_shape=jax.ShapeDtypeStruct((G, 1), _f32),
    )(u, W1, b1, W2, b2)


def kernel(x, edge_attr, params, edge_index, batch):
    row3 = edge_index[0].reshape(NW, NCH, CH)
    col3 = edge_index[1].reshape(NW, NCH, CH)
    brow = _sc_take1d(batch, row3, jnp.int32)
    batch_r = batch.reshape(NB_N, 1, BN)
    brow_r = brow.reshape(NB_E, 1, BE)
    zrows = jnp.zeros((SPT, H), _f32)

    def r1(v):
        return v.reshape(1, -1)

    pn = params["mlp_node"]
    pe = params["mlp_edge"]
    pg = params["mlp_global"]
    x1, S_x, Q_x, cntn = _enc_x(x, batch_r, pn["W1"], r1(pn["b1"]),
                                pn["W2"], r1(pn["b2"]))
    e0, S_raw, S_e, Q_e, cnte = _enc_e(edge_attr, brow_r, pe["W1"], r1(pe["b1"]),
                                       pe["W2"], r1(pe["b2"]))
    bnx = params["bn_node"][DEPTH]
    bne = params["bn_edge"][DEPTH]
    bnu = params["bn_global"][DEPTH]
    u, scal, svec = _glob0(S_raw, cnte, cntn, S_x, Q_x, S_e, Q_e,
                           r1(bnx["g"]), r1(bnx["b"]), r1(bne["g"]), r1(bne["b"]),
                           r1(bnu["g"]), r1(bnu["b"]),
                           pg["W1"], r1(pg["b1"]), pg["W2"], r1(pg["b2"]))

    eprev = e0
    xraw = x1
    for i in range(DEPTH):
        lp = params["layers"][i]
        em = lp["edge_mlp"]
        W1 = em["W1"]
        xnorm, t, xb = _tables(xraw, batch_r, scal, svec, u,
                               W1[:H], W1[H:2 * H], W1[3 * H:], r1(em["b1"]))
        t_row, xb_col = _sc_gather(row3, col3, t, xb)
        enew, S_e, Q_e = _edge(t_row, xb_col, eprev, brow_r, svec, scal,
                               W1[2 * H:3 * H], em["W2"], r1(em["b2"]))
        aggA, aggB = _sc_scatter(row3, col3, xnorm, enew, zrows)
        nm1 = lp["node_mlp_1"]
        nm2 = lp["node_mlp_2"]
        xnew, S_xn, Q_xn = _node(aggA, aggB, xnorm, batch_r, u,
                                 nm1["W1"], r1(nm1["b1"]), nm1["W2"], r1(nm1["b2"]),
                                 nm2["W1"], r1(nm2["b1"]), nm2["W2"], r1(nm2["b2"]))
        bnx = params["bn_node"][i]
        bne = params["bn_edge"][i]
        bnu = params["bn_global"][i]
        gm = lp["global_mlp"]
        u, scal = _glob_layer(u, S_e, Q_e, S_xn, Q_xn, svec,
                              r1(bnx["g"]), r1(bnx["b"]), r1(bne["g"]), r1(bne["b"]),
                              r1(bnu["g"]), r1(bnu["b"]),
                              gm["W1"], r1(gm["b1"]), gm["W2"], r1(gm["b2"]))
        eprev = enew
        xraw = xnew

    m1 = params["mlp1"]
    return _final(u, m1["W1"], r1(m1["b1"]), m1["W2"], r1(m1["b2"]))
